# Initial kernel scaffold; baseline (speedup 1.0000x reference)
import jax
import jax.numpy as jnp
from jax.experimental import pallas as pl


def kernel(x, edge_index, W_self0, W_neigh0, b0, W_self1, W_neigh1, b1):
    raise NotImplementedError("write your pallas kernel here")


# speedup vs baseline: 6.5117x; 6.5117x over previous
import jax
import jax.numpy as jnp
from jax import lax
from jax.experimental import pallas as pl
from jax.experimental.pallas import tpu as pltpu
from jax.experimental.pallas import tpu_sc as plsc

N = 10000
NC = 2
NS = 16
NW = NC * NS
CB = 128
NP = 10112
ZROWS = NP // NS
KG = 8


def _make_agg(K, D):
  mesh = plsc.VectorSubcoreMesh(core_axis_name="c", subcore_axis_name="s")
  out_type = [jax.ShapeDtypeStruct((NC, NP, D), jnp.float32)]
  scratch = [
      pltpu.VMEM((KG, CB), jnp.int32),
      pltpu.VMEM((KG, CB), jnp.int32),
      pltpu.VMEM((CB, D), jnp.float32),
      pltpu.VMEM_SHARED((NP, D), jnp.float32),
  ]

  def body(h_hbm, srcs_hbm, dsts_hbm, zeros_hbm, part_hbm,
           src_v, dst_v, rows_v, acc_sh):
    c = lax.axis_index("c")
    s = lax.axis_index("s")
    wid = c * NS + s

    zbase = s * ZROWS
    nfull, rem = ZROWS // CB, ZROWS % CB
    pltpu.sync_copy(zeros_hbm, rows_v)
    for i in range(nfull):
      pltpu.sync_copy(rows_v, acc_sh.at[pl.ds(zbase + i * CB, CB)])
    if rem:
      off = zbase + nfull * CB
      pltpu.sync_copy(rows_v.at[pl.ds(0, rem)], acc_sh.at[pl.ds(off, rem)])

    plsc.subcore_barrier()

    def group(g, _):
      pltpu.sync_copy(srcs_hbm.at[wid, pl.ds(g * KG, KG)], src_v)
      pltpu.sync_copy(dsts_hbm.at[wid, pl.ds(g * KG, KG)], dst_v)
      for j in range(KG):
        pltpu.sync_copy(h_hbm.at[src_v.at[j]], rows_v)
        pltpu.sync_copy(rows_v, acc_sh.at[dst_v.at[j]], add=True)
      return 0
    lax.fori_loop(0, K // KG, group, 0)

    plsc.subcore_barrier()

    for i in range(nfull):
      off = zbase + i * CB
      pltpu.sync_copy(acc_sh.at[pl.ds(off, CB)], rows_v)
      pltpu.sync_copy(rows_v, part_hbm.at[c, pl.ds(off, CB)])
    if rem:
      off = zbase + nfull * CB
      pltpu.sync_copy(acc_sh.at[pl.ds(off, rem)], rows_v.at[pl.ds(0, rem)])
      pltpu.sync_copy(rows_v.at[pl.ds(0, rem)], part_hbm.at[c, pl.ds(off, rem)])

  return pl.kernel(body, out_type=out_type, mesh=mesh, scratch_types=scratch)


def _make_deg(K, D):
  mesh = plsc.VectorSubcoreMesh(core_axis_name="c", subcore_axis_name="s")
  out_type = [jax.ShapeDtypeStruct((NC, NP, D), jnp.float32)]
  scratch = [
      pltpu.VMEM((KG, CB), jnp.int32),
      pltpu.VMEM((CB, D), jnp.float32),
      pltpu.VMEM_SHARED((NP, D), jnp.float32),
  ]

  def body(dsts_hbm, onez_hbm, deg_hbm, dst_v, rows_v, acc_sh):
    c = lax.axis_index("c")
    s = lax.axis_index("s")
    wid = c * NS + s

    zbase = s * ZROWS
    nfull, rem = ZROWS // CB, ZROWS % CB
    pltpu.sync_copy(onez_hbm.at[pl.ds(CB, CB)], rows_v)
    for i in range(nfull):
      pltpu.sync_copy(rows_v, acc_sh.at[pl.ds(zbase + i * CB, CB)])
    if rem:
      off = zbase + nfull * CB
      pltpu.sync_copy(rows_v.at[pl.ds(0, rem)], acc_sh.at[pl.ds(off, rem)])
    pltpu.sync_copy(onez_hbm.at[pl.ds(0, CB)], rows_v)

    plsc.subcore_barrier()

    def group(g, _):
      pltpu.sync_copy(dsts_hbm.at[wid, pl.ds(g * KG, KG)], dst_v)
      for j in range(KG):
        pltpu.sync_copy(rows_v, acc_sh.at[dst_v.at[j]], add=True)
      return 0
    lax.fori_loop(0, K // KG, group, 0)

    plsc.subcore_barrier()

    for i in range(nfull):
      off = zbase + i * CB
      pltpu.sync_copy(acc_sh.at[pl.ds(off, CB)], rows_v)
      pltpu.sync_copy(rows_v, deg_hbm.at[c, pl.ds(off, CB)])
    if rem:
      off = zbase + nfull * CB
      pltpu.sync_copy(acc_sh.at[pl.ds(off, rem)], rows_v.at[pl.ds(0, rem)])
      pltpu.sync_copy(rows_v.at[pl.ds(0, rem)], deg_hbm.at[c, pl.ds(off, rem)])

  return pl.kernel(body, out_type=out_type, mesh=mesh, scratch_types=scratch)


def _tc_layer1(x, parts, degs, W_self, W_neigh, b):
  Nn, Din = x.shape
  Dout = W_self.shape[1]
  R = 400
  grid = (Nn // R,)

  def body(x_ref, p_ref, d_ref, ws_ref, wn_ref, b_ref, h_ref, dinv_ref):
    p = p_ref[0] + p_ref[1]
    deg = d_ref[0, :, 0:1] + d_ref[1, :, 0:1]
    dinv = 1.0 / jnp.maximum(deg, 1.0)
    mean = p * dinv
    h = (jnp.dot(x_ref[...], ws_ref[...], preferred_element_type=jnp.float32)
         + jnp.dot(mean, wn_ref[...], preferred_element_type=jnp.float32)
         + b_ref[...])
    h_ref[...] = jnp.maximum(h, 0.0)
    dinv_ref[...] = jnp.broadcast_to(dinv, (R, 8))

  return pl.pallas_call(
      body,
      grid=grid,
      in_specs=[
          pl.BlockSpec((R, Din), lambda i: (i, 0)),
          pl.BlockSpec((NC, R, Din), lambda i: (0, i, 0)),
          pl.BlockSpec((NC, R, 128), lambda i: (0, i, 0)),
          pl.BlockSpec((Din, Dout), lambda i: (0, 0)),
          pl.BlockSpec((Din, Dout), lambda i: (0, 0)),
          pl.BlockSpec((1, Dout), lambda i: (0, 0)),
      ],
      out_specs=[
          pl.BlockSpec((R, Dout), lambda i: (i, 0)),
          pl.BlockSpec((R, 8), lambda i: (i, 0)),
      ],
      out_shape=[
          jax.ShapeDtypeStruct((Nn, Dout), jnp.float32),
          jax.ShapeDtypeStruct((Nn, 8), jnp.float32),
      ],
  )(x, parts, degs, W_self, W_neigh, b.reshape(1, Dout))


def _tc_layer2(h, parts, dinv, W_self, W_neigh, b):
  Nn, Din = h.shape
  Dout = W_self.shape[1]
  R = 400
  grid = (Nn // R,)

  def body(h_ref, p_ref, d_ref, ws_ref, wn_ref, b_ref, o_ref):
    mean = (p_ref[0] + p_ref[1]) * d_ref[:, 0:1]
    o_ref[...] = (
        jnp.dot(h_ref[...], ws_ref[...], preferred_element_type=jnp.float32)
        + jnp.dot(mean, wn_ref[...], preferred_element_type=jnp.float32)
        + b_ref[...])

  return pl.pallas_call(
      body,
      grid=grid,
      in_specs=[
          pl.BlockSpec((R, Din), lambda i: (i, 0)),
          pl.BlockSpec((NC, R, Din), lambda i: (0, i, 0)),
          pl.BlockSpec((R, 8), lambda i: (i, 0)),
          pl.BlockSpec((Din, Dout), lambda i: (0, 0)),
          pl.BlockSpec((Din, Dout), lambda i: (0, 0)),
          pl.BlockSpec((1, Dout), lambda i: (0, 0)),
      ],
      out_specs=pl.BlockSpec((R, Dout), lambda i: (i, 0)),
      out_shape=jax.ShapeDtypeStruct((Nn, Dout), jnp.float32),
  )(h, parts, dinv, W_self, W_neigh, b.reshape(1, Dout))


def kernel(x, edge_index, W_self0, W_neigh0, b0, W_self1, W_neigh1, b1):
  src = edge_index[0]
  dst = edge_index[1]
  E = src.shape[0]
  K = -(-E // (NW * CB * KG)) * KG
  pad = NW * K * CB - E
  it = jnp.arange(pad, dtype=jnp.int32)
  src_p = jnp.concatenate([src, it % jnp.int32(x.shape[0])])
  dst_p = jnp.concatenate([dst, N + it % jnp.int32(NP - N)])
  srcs = src_p.reshape(NW, K, CB)
  dsts = dst_p.reshape(NW, K, CB)

  D = x.shape[1]
  zeros = jnp.zeros((CB, D), jnp.float32)
  onez = jnp.concatenate([jnp.ones((CB, 128), jnp.float32),
                          jnp.zeros((CB, 128), jnp.float32)])
  (degs,) = _make_deg(K, 128)(dsts, onez)
  (parts0,) = _make_agg(K, D)(x, srcs, dsts, zeros)
  h, dinv = _tc_layer1(x, parts0, degs, W_self0, W_neigh0, b0)
  (parts1,) = _make_agg(K, D)(h, srcs, dsts, zeros)
  out = _tc_layer2(h, parts1, dinv, W_self1, W_neigh1, b1)
  return out


# speedup vs baseline: 7.7327x; 1.1875x over previous
import jax
import jax.numpy as jnp
from jax import lax
from jax.experimental import pallas as pl
from jax.experimental.pallas import tpu as pltpu
from jax.experimental.pallas import tpu_sc as plsc

N = 10000
NC = 2
NS = 16
NW = NC * NS
CB = 128
NP = 10112
ZROWS = NP // NS
KG = 8


def _make_agg(K, D):
  mesh = plsc.VectorSubcoreMesh(core_axis_name="c", subcore_axis_name="s")
  out_type = [jax.ShapeDtypeStruct((NC, NP, D), jnp.float32)]
  scratch = [
      pltpu.VMEM((KG, CB), jnp.int32),
      pltpu.VMEM((KG, CB), jnp.int32),
      pltpu.VMEM((CB, D), jnp.float32),
      pltpu.VMEM((CB, D), jnp.float32),
      pltpu.VMEM_SHARED((NP, D), jnp.float32),
      pltpu.SemaphoreType.DMA,
      pltpu.SemaphoreType.DMA,
      pltpu.SemaphoreType.DMA,
      pltpu.SemaphoreType.DMA,
  ]

  def body(h_hbm, srcs_hbm, dsts_hbm, zeros_hbm, part_hbm,
           src_v, dst_v, rows_v, rows2_v, acc_sh, gsa, gsb, ssa, ssb):
    c = lax.axis_index("c")
    s = lax.axis_index("s")
    wid = c * NS + s

    zbase = s * ZROWS
    nfull, rem = ZROWS // CB, ZROWS % CB
    pltpu.sync_copy(zeros_hbm, rows_v)
    for i in range(nfull):
      pltpu.sync_copy(rows_v, acc_sh.at[pl.ds(zbase + i * CB, CB)])
    if rem:
      off = zbase + nfull * CB
      pltpu.sync_copy(rows_v.at[pl.ds(0, rem)], acc_sh.at[pl.ds(off, rem)])

    plsc.subcore_barrier()

    bufs = (rows_v, rows2_v)
    gsems = (gsa, gsb)
    ssems = (ssa, ssb)

    def group(g, _):
      pltpu.sync_copy(srcs_hbm.at[wid, pl.ds(g * KG, KG)], src_v)
      pltpu.sync_copy(dsts_hbm.at[wid, pl.ds(g * KG, KG)], dst_v)
      gd = [None] * KG
      sd = [None] * KG
      gd[0] = pltpu.make_async_copy(h_hbm.at[src_v.at[0]], bufs[0], gsems[0])
      gd[0].start()
      for j in range(KG):
        gd[j].wait()
        sd[j] = pltpu.make_async_copy(bufs[j % 2], acc_sh.at[dst_v.at[j]],
                                      ssems[j % 2])
        sd[j].start(add=True)
        if j + 1 < KG:
          if j >= 1:
            sd[j - 1].wait()
          nb = (j + 1) % 2
          gd[j + 1] = pltpu.make_async_copy(h_hbm.at[src_v.at[j + 1]],
                                            bufs[nb], gsems[nb])
          gd[j + 1].start()
      if KG >= 2:
        sd[KG - 2].wait()
      sd[KG - 1].wait()
      return 0
    lax.fori_loop(0, K // KG, group, 0)

    plsc.subcore_barrier()

    for i in range(nfull):
      off = zbase + i * CB
      pltpu.sync_copy(acc_sh.at[pl.ds(off, CB)], rows_v)
      pltpu.sync_copy(rows_v, part_hbm.at[c, pl.ds(off, CB)])
    if rem:
      off = zbase + nfull * CB
      pltpu.sync_copy(acc_sh.at[pl.ds(off, rem)], rows_v.at[pl.ds(0, rem)])
      pltpu.sync_copy(rows_v.at[pl.ds(0, rem)], part_hbm.at[c, pl.ds(off, rem)])

  return pl.kernel(body, out_type=out_type, mesh=mesh, scratch_types=scratch)


def _make_deg(K, D):
  mesh = plsc.VectorSubcoreMesh(core_axis_name="c", subcore_axis_name="s")
  out_type = [jax.ShapeDtypeStruct((NC, NP, D), jnp.float32)]
  scratch = [
      pltpu.VMEM((KG, CB), jnp.int32),
      pltpu.VMEM((CB, D), jnp.float32),
      pltpu.VMEM_SHARED((NP, D), jnp.float32),
      pltpu.SemaphoreType.DMA,
  ]

  def body(dsts_hbm, onez_hbm, deg_hbm, dst_v, rows_v, acc_sh, ssem):
    c = lax.axis_index("c")
    s = lax.axis_index("s")
    wid = c * NS + s

    zbase = s * ZROWS
    nfull, rem = ZROWS // CB, ZROWS % CB
    pltpu.sync_copy(onez_hbm.at[pl.ds(CB, CB)], rows_v)
    for i in range(nfull):
      pltpu.sync_copy(rows_v, acc_sh.at[pl.ds(zbase + i * CB, CB)])
    if rem:
      off = zbase + nfull * CB
      pltpu.sync_copy(rows_v.at[pl.ds(0, rem)], acc_sh.at[pl.ds(off, rem)])
    pltpu.sync_copy(onez_hbm.at[pl.ds(0, CB)], rows_v)

    plsc.subcore_barrier()

    def group(g, _):
      pltpu.sync_copy(dsts_hbm.at[wid, pl.ds(g * KG, KG)], dst_v)
      sd = []
      for j in range(KG):
        d = pltpu.make_async_copy(rows_v, acc_sh.at[dst_v.at[j]], ssem)
        d.start(add=True)
        sd.append(d)
      for d in sd:
        d.wait()
      return 0
    lax.fori_loop(0, K // KG, group, 0)

    plsc.subcore_barrier()

    for i in range(nfull):
      off = zbase + i * CB
      pltpu.sync_copy(acc_sh.at[pl.ds(off, CB)], rows_v)
      pltpu.sync_copy(rows_v, deg_hbm.at[c, pl.ds(off, CB)])
    if rem:
      off = zbase + nfull * CB
      pltpu.sync_copy(acc_sh.at[pl.ds(off, rem)], rows_v.at[pl.ds(0, rem)])
      pltpu.sync_copy(rows_v.at[pl.ds(0, rem)], deg_hbm.at[c, pl.ds(off, rem)])

  return pl.kernel(body, out_type=out_type, mesh=mesh, scratch_types=scratch)


def _tc_layer1(x, parts, degs, W_self, W_neigh, b):
  Nn, Din = x.shape
  Dout = W_self.shape[1]
  R = 400
  grid = (Nn // R,)

  def body(x_ref, p_ref, d_ref, ws_ref, wn_ref, b_ref, h_ref, dinv_ref):
    p = p_ref[0] + p_ref[1]
    deg = d_ref[0, :, 0:1] + d_ref[1, :, 0:1]
    dinv = 1.0 / jnp.maximum(deg, 1.0)
    mean = p * dinv
    h = (jnp.dot(x_ref[...], ws_ref[...], preferred_element_type=jnp.float32)
         + jnp.dot(mean, wn_ref[...], preferred_element_type=jnp.float32)
         + b_ref[...])
    h_ref[...] = jnp.maximum(h, 0.0)
    dinv_ref[...] = jnp.broadcast_to(dinv, (R, 8))

  return pl.pallas_call(
      body,
      grid=grid,
      in_specs=[
          pl.BlockSpec((R, Din), lambda i: (i, 0)),
          pl.BlockSpec((NC, R, Din), lambda i: (0, i, 0)),
          pl.BlockSpec((NC, R, 128), lambda i: (0, i, 0)),
          pl.BlockSpec((Din, Dout), lambda i: (0, 0)),
          pl.BlockSpec((Din, Dout), lambda i: (0, 0)),
          pl.BlockSpec((1, Dout), lambda i: (0, 0)),
      ],
      out_specs=[
          pl.BlockSpec((R, Dout), lambda i: (i, 0)),
          pl.BlockSpec((R, 8), lambda i: (i, 0)),
      ],
      out_shape=[
          jax.ShapeDtypeStruct((Nn, Dout), jnp.float32),
          jax.ShapeDtypeStruct((Nn, 8), jnp.float32),
      ],
  )(x, parts, degs, W_self, W_neigh, b.reshape(1, Dout))


def _tc_layer2(h, parts, dinv, W_self, W_neigh, b):
  Nn, Din = h.shape
  Dout = W_self.shape[1]
  R = 400
  grid = (Nn // R,)

  def body(h_ref, p_ref, d_ref, ws_ref, wn_ref, b_ref, o_ref):
    mean = (p_ref[0] + p_ref[1]) * d_ref[:, 0:1]
    o_ref[...] = (
        jnp.dot(h_ref[...], ws_ref[...], preferred_element_type=jnp.float32)
        + jnp.dot(mean, wn_ref[...], preferred_element_type=jnp.float32)
        + b_ref[...])

  return pl.pallas_call(
      body,
      grid=grid,
      in_specs=[
          pl.BlockSpec((R, Din), lambda i: (i, 0)),
          pl.BlockSpec((NC, R, Din), lambda i: (0, i, 0)),
          pl.BlockSpec((R, 8), lambda i: (i, 0)),
          pl.BlockSpec((Din, Dout), lambda i: (0, 0)),
          pl.BlockSpec((Din, Dout), lambda i: (0, 0)),
          pl.BlockSpec((1, Dout), lambda i: (0, 0)),
      ],
      out_specs=pl.BlockSpec((R, Dout), lambda i: (i, 0)),
      out_shape=jax.ShapeDtypeStruct((Nn, Dout), jnp.float32),
  )(h, parts, dinv, W_self, W_neigh, b.reshape(1, Dout))


def kernel(x, edge_index, W_self0, W_neigh0, b0, W_self1, W_neigh1, b1):
  src = edge_index[0]
  dst = edge_index[1]
  E = src.shape[0]
  K = -(-E // (NW * CB * KG)) * KG
  pad = NW * K * CB - E
  it = jnp.arange(pad, dtype=jnp.int32)
  src_p = jnp.concatenate([src, it % jnp.int32(x.shape[0])])
  dst_p = jnp.concatenate([dst, N + it % jnp.int32(NP - N)])
  srcs = src_p.reshape(NW, K, CB)
  dsts = dst_p.reshape(NW, K, CB)

  D = x.shape[1]
  zeros = jnp.zeros((CB, D), jnp.float32)
  onez = jnp.concatenate([jnp.ones((CB, 128), jnp.float32),
                          jnp.zeros((CB, 128), jnp.float32)])
  (degs,) = _make_deg(K, 128)(dsts, onez)
  (parts0,) = _make_agg(K, D)(x, srcs, dsts, zeros)
  h, dinv = _tc_layer1(x, parts0, degs, W_self0, W_neigh0, b0)
  (parts1,) = _make_agg(K, D)(h, srcs, dsts, zeros)
  out = _tc_layer2(h, parts1, dinv, W_self1, W_neigh1, b1)
  return out


# speedup vs baseline: 8.4408x; 1.0916x over previous
import jax
import jax.numpy as jnp
from jax import lax
from jax.experimental import pallas as pl
from jax.experimental.pallas import tpu as pltpu
from jax.experimental.pallas import tpu_sc as plsc

N = 10000
NC = 2
NS = 16
NW = NC * NS
CB = 128
NP = 10112
ZROWS = NP // NS
KG = 8
DEGW = 32


def _make_agg(K, D):
  mesh = plsc.VectorSubcoreMesh(core_axis_name="c", subcore_axis_name="s")
  out_type = [jax.ShapeDtypeStruct((NC, NP, D), jnp.float32)]
  scratch = [
      pltpu.VMEM((KG, CB), jnp.int32),
      pltpu.VMEM((KG, CB), jnp.int32),
      pltpu.VMEM((CB, D), jnp.float32),
      pltpu.VMEM((CB, D), jnp.float32),
      pltpu.VMEM_SHARED((NP, D), jnp.float32),
      pltpu.SemaphoreType.DMA,
      pltpu.SemaphoreType.DMA,
      pltpu.SemaphoreType.DMA,
      pltpu.SemaphoreType.DMA,
  ]

  def body(h_hbm, srcs_hbm, dsts_hbm, zeros_hbm, part_hbm,
           src_v, dst_v, rows_v, rows2_v, acc_sh, gsa, gsb, ssa, ssb):
    c = lax.axis_index("c")
    s = lax.axis_index("s")
    wid = c * NS + s

    zbase = s * ZROWS
    nfull, rem = ZROWS // CB, ZROWS % CB
    pltpu.sync_copy(zeros_hbm, rows_v)
    for i in range(nfull):
      pltpu.sync_copy(rows_v, acc_sh.at[pl.ds(zbase + i * CB, CB)])
    if rem:
      off = zbase + nfull * CB
      pltpu.sync_copy(rows_v.at[pl.ds(0, rem)], acc_sh.at[pl.ds(off, rem)])

    plsc.subcore_barrier()

    bufs = (rows_v, rows2_v)
    gsems = (gsa, gsb)
    ssems = (ssa, ssb)

    def group(g, _):
      pltpu.sync_copy(srcs_hbm.at[wid, pl.ds(g * KG, KG)], src_v)
      pltpu.sync_copy(dsts_hbm.at[wid, pl.ds(g * KG, KG)], dst_v)
      gd = [None] * KG
      sd = [None] * KG
      gd[0] = pltpu.make_async_copy(h_hbm.at[src_v.at[0]], bufs[0], gsems[0])
      gd[0].start()
      for j in range(KG):
        gd[j].wait()
        sd[j] = pltpu.make_async_copy(bufs[j % 2], acc_sh.at[dst_v.at[j]],
                                      ssems[j % 2])
        sd[j].start(add=True)
        if j + 1 < KG:
          if j >= 1:
            sd[j - 1].wait()
          nb = (j + 1) % 2
          gd[j + 1] = pltpu.make_async_copy(h_hbm.at[src_v.at[j + 1]],
                                            bufs[nb], gsems[nb])
          gd[j + 1].start()
      if KG >= 2:
        sd[KG - 2].wait()
      sd[KG - 1].wait()
      return 0
    lax.fori_loop(0, K // KG, group, 0)

    plsc.subcore_barrier()

    for i in range(nfull):
      off = zbase + i * CB
      pltpu.sync_copy(acc_sh.at[pl.ds(off, CB)], rows_v)
      pltpu.sync_copy(rows_v, part_hbm.at[c, pl.ds(off, CB)])
    if rem:
      off = zbase + nfull * CB
      pltpu.sync_copy(acc_sh.at[pl.ds(off, rem)], rows_v.at[pl.ds(0, rem)])
      pltpu.sync_copy(rows_v.at[pl.ds(0, rem)], part_hbm.at[c, pl.ds(off, rem)])

  return pl.kernel(body, out_type=out_type, mesh=mesh, scratch_types=scratch)


def _make_deg(K, D):
  mesh = plsc.VectorSubcoreMesh(core_axis_name="c", subcore_axis_name="s")
  out_type = [jax.ShapeDtypeStruct((NC, NP, D), jnp.float32)]
  scratch = [
      pltpu.VMEM((KG, CB), jnp.int32),
      pltpu.VMEM((CB, D), jnp.float32),
      pltpu.VMEM_SHARED((NP, D), jnp.float32),
      pltpu.SemaphoreType.DMA,
  ]

  def body(dsts_hbm, onez_hbm, deg_hbm, dst_v, rows_v, acc_sh, ssem):
    c = lax.axis_index("c")
    s = lax.axis_index("s")
    wid = c * NS + s

    zbase = s * ZROWS
    nfull, rem = ZROWS // CB, ZROWS % CB
    pltpu.sync_copy(onez_hbm.at[pl.ds(CB, CB)], rows_v)
    for i in range(nfull):
      pltpu.sync_copy(rows_v, acc_sh.at[pl.ds(zbase + i * CB, CB)])
    if rem:
      off = zbase + nfull * CB
      pltpu.sync_copy(rows_v.at[pl.ds(0, rem)], acc_sh.at[pl.ds(off, rem)])
    pltpu.sync_copy(onez_hbm.at[pl.ds(0, CB)], rows_v)

    plsc.subcore_barrier()

    def group(g, _):
      pltpu.sync_copy(dsts_hbm.at[wid, pl.ds(g * KG, KG)], dst_v)
      sd = []
      for j in range(KG):
        d = pltpu.make_async_copy(rows_v, acc_sh.at[dst_v.at[j]], ssem)
        d.start(add=True)
        sd.append(d)
      for d in sd:
        d.wait()
      return 0
    lax.fori_loop(0, K // KG, group, 0)

    plsc.subcore_barrier()

    for i in range(nfull):
      off = zbase + i * CB
      pltpu.sync_copy(acc_sh.at[pl.ds(off, CB)], rows_v)
      pltpu.sync_copy(rows_v, deg_hbm.at[c, pl.ds(off, CB)])
    if rem:
      off = zbase + nfull * CB
      pltpu.sync_copy(acc_sh.at[pl.ds(off, rem)], rows_v.at[pl.ds(0, rem)])
      pltpu.sync_copy(rows_v.at[pl.ds(0, rem)], deg_hbm.at[c, pl.ds(off, rem)])

  return pl.kernel(body, out_type=out_type, mesh=mesh, scratch_types=scratch)


def _tc_layer1(x, parts, degs, W_self, W_neigh, b):
  Nn, Din = x.shape
  Dout = W_self.shape[1]
  R = 400
  grid = (Nn // R,)

  def body(x_ref, p_ref, d_ref, ws_ref, wn_ref, b_ref, h_ref, dinv_ref):
    p = p_ref[0] + p_ref[1]
    deg = d_ref[0, :, 0:1] + d_ref[1, :, 0:1]
    dinv = 1.0 / jnp.maximum(deg, 1.0)
    mean = p * dinv
    h = (jnp.dot(x_ref[...], ws_ref[...], preferred_element_type=jnp.float32)
         + jnp.dot(mean, wn_ref[...], preferred_element_type=jnp.float32)
         + b_ref[...])
    h_ref[...] = jnp.maximum(h, 0.0)
    dinv_ref[...] = jnp.broadcast_to(dinv, (R, 8))

  return pl.pallas_call(
      body,
      grid=grid,
      in_specs=[
          pl.BlockSpec((R, Din), lambda i: (i, 0)),
          pl.BlockSpec((NC, R, Din), lambda i: (0, i, 0)),
          pl.BlockSpec((NC, R, DEGW), lambda i: (0, i, 0)),
          pl.BlockSpec((Din, Dout), lambda i: (0, 0)),
          pl.BlockSpec((Din, Dout), lambda i: (0, 0)),
          pl.BlockSpec((1, Dout), lambda i: (0, 0)),
      ],
      out_specs=[
          pl.BlockSpec((R, Dout), lambda i: (i, 0)),
          pl.BlockSpec((R, 8), lambda i: (i, 0)),
      ],
      out_shape=[
          jax.ShapeDtypeStruct((Nn, Dout), jnp.float32),
          jax.ShapeDtypeStruct((Nn, 8), jnp.float32),
      ],
  )(x, parts, degs, W_self, W_neigh, b.reshape(1, Dout))


def _tc_layer2(h, parts, dinv, W_self, W_neigh, b):
  Nn, Din = h.shape
  Dout = W_self.shape[1]
  R = 400
  grid = (Nn // R,)

  def body(h_ref, p_ref, d_ref, ws_ref, wn_ref, b_ref, o_ref):
    mean = (p_ref[0] + p_ref[1]) * d_ref[:, 0:1]
    o_ref[...] = (
        jnp.dot(h_ref[...], ws_ref[...], preferred_element_type=jnp.float32)
        + jnp.dot(mean, wn_ref[...], preferred_element_type=jnp.float32)
        + b_ref[...])

  return pl.pallas_call(
      body,
      grid=grid,
      in_specs=[
          pl.BlockSpec((R, Din), lambda i: (i, 0)),
          pl.BlockSpec((NC, R, Din), lambda i: (0, i, 0)),
          pl.BlockSpec((R, 8), lambda i: (i, 0)),
          pl.BlockSpec((Din, Dout), lambda i: (0, 0)),
          pl.BlockSpec((Din, Dout), lambda i: (0, 0)),
          pl.BlockSpec((1, Dout), lambda i: (0, 0)),
      ],
      out_specs=pl.BlockSpec((R, Dout), lambda i: (i, 0)),
      out_shape=jax.ShapeDtypeStruct((Nn, Dout), jnp.float32),
  )(h, parts, dinv, W_self, W_neigh, b.reshape(1, Dout))


def kernel(x, edge_index, W_self0, W_neigh0, b0, W_self1, W_neigh1, b1):
  src = edge_index[0]
  dst = edge_index[1]
  E = src.shape[0]
  K = -(-E // (NW * CB * KG)) * KG
  pad = NW * K * CB - E
  it = jnp.arange(pad, dtype=jnp.int32)
  src_p = jnp.concatenate([src, it % jnp.int32(x.shape[0])])
  dst_p = jnp.concatenate([dst, N + it % jnp.int32(NP - N)])
  srcs = src_p.reshape(NW, K, CB)
  dsts = dst_p.reshape(NW, K, CB)

  D = x.shape[1]
  zeros = jnp.zeros((CB, D), jnp.float32)
  onez = jnp.concatenate([jnp.ones((CB, DEGW), jnp.float32),
                          jnp.zeros((CB, DEGW), jnp.float32)])
  (degs,) = _make_deg(K, DEGW)(dsts, onez)
  (parts0,) = _make_agg(K, D)(x, srcs, dsts, zeros)
  h, dinv = _tc_layer1(x, parts0, degs, W_self0, W_neigh0, b0)
  (parts1,) = _make_agg(K, D)(h, srcs, dsts, zeros)
  out = _tc_layer2(h, parts1, dinv, W_self1, W_neigh1, b1)
  return out
